# baseline (speedup 1.0000x reference)
import functools

import jax
import jax.numpy as jnp
from jax import lax
from jax.experimental import pallas as pl
from jax.experimental.pallas import tpu as pltpu
from jax.experimental.pallas import tpu_sc as plsc

N = 10000
E = 320000
NPAD = 10112
TILES = 16
TRANS = 128
JSTEPS = 160
EPT = TRANS * JSTEPS
EPAD = EPT * TILES
ZROWS = NPAD // TILES
GB = 8


def _sc_gather_scatter(table, gat2d, dst2d, z128):
    mesh = plsc.VectorSubcoreMesh(core_axis_name="c", subcore_axis_name="s",
                                  num_cores=1)

    @functools.partial(
        pl.kernel,
        mesh=mesh,
        out_type=jax.ShapeDtypeStruct((NPAD, 128), jnp.float32),
        scratch_types=[
            pltpu.VMEM((GB, TRANS), jnp.int32),
            pltpu.VMEM((GB, TRANS), jnp.int32),
            pltpu.VMEM((2, TRANS, 128), jnp.float32),
            pltpu.VMEM_SHARED((NPAD, 128), jnp.float32),
            pltpu.SemaphoreType.DMA,
            pltpu.SemaphoreType.DMA,
        ],
    )
    def k(tab_hbm, gat_hbm, dst_hbm, z128_hbm, out_hbm,
          gidx, didx, rows, acc, sg, ss):
        s = lax.axis_index("s")
        pltpu.sync_copy(z128_hbm, acc.at[pl.ds(s * ZROWS, ZROWS)])
        plsc.subcore_barrier()

        def body(g, carry):
            grow = s * JSTEPS + g * GB
            pltpu.sync_copy(gat_hbm.at[pl.ds(grow, GB)], gidx)
            pltpu.sync_copy(dst_hbm.at[pl.ds(grow, GB)], didx)
            hg = {}
            hsc = {}
            hg[0] = pltpu.async_copy(tab_hbm.at[gidx.at[0]], rows.at[0], sg)
            hg[1] = pltpu.async_copy(tab_hbm.at[gidx.at[1]], rows.at[1], sg)
            for j in range(GB):
                if j >= 1:
                    hsc[j - 1].wait()
                if 1 <= j and j + 1 < GB:
                    hg[j + 1] = pltpu.async_copy(
                        tab_hbm.at[gidx.at[j + 1]], rows.at[(j + 1) % 2], sg)
                hg[j].wait()
                hsc[j] = pltpu.async_copy(
                    rows.at[j % 2], acc.at[didx.at[j]], ss, add=True)
            hsc[GB - 1].wait()
            return carry

        lax.fori_loop(0, JSTEPS // GB, body, 0)
        plsc.subcore_barrier()
        pltpu.sync_copy(acc.at[pl.ds(s * ZROWS, ZROWS)],
                        out_hbm.at[pl.ds(s * ZROWS, ZROWS)])

    return k(table, gat2d, dst2d, z128)


def _sc_linear_scatter(ep, dst2d, z128):
    mesh = plsc.VectorSubcoreMesh(core_axis_name="c", subcore_axis_name="s",
                                  num_cores=1)

    @functools.partial(
        pl.kernel,
        mesh=mesh,
        out_type=jax.ShapeDtypeStruct((NPAD, 128), jnp.float32),
        scratch_types=[
            pltpu.VMEM((GB, TRANS), jnp.int32),
            pltpu.VMEM((2, TRANS, 128), jnp.float32),
            pltpu.VMEM_SHARED((NPAD, 128), jnp.float32),
            pltpu.SemaphoreType.DMA,
            pltpu.SemaphoreType.DMA,
        ],
    )
    def k(ep_hbm, dst_hbm, z128_hbm, out_hbm, didx, rows, acc, sg, ss):
        s = lax.axis_index("s")
        pltpu.sync_copy(z128_hbm, acc.at[pl.ds(s * ZROWS, ZROWS)])
        plsc.subcore_barrier()

        def body(g, carry):
            grow = s * JSTEPS + g * GB
            pltpu.sync_copy(dst_hbm.at[pl.ds(grow, GB)], didx)
            base = s * EPT + g * GB * TRANS
            hg = {}
            hsc = {}
            hg[0] = pltpu.async_copy(
                ep_hbm.at[pl.ds(base, TRANS)], rows.at[0], sg)
            hg[1] = pltpu.async_copy(
                ep_hbm.at[pl.ds(base + TRANS, TRANS)], rows.at[1], sg)
            for j in range(GB):
                if j >= 1:
                    hsc[j - 1].wait()
                if 1 <= j and j + 1 < GB:
                    hg[j + 1] = pltpu.async_copy(
                        ep_hbm.at[pl.ds(base + (j + 1) * TRANS, TRANS)],
                        rows.at[(j + 1) % 2], sg)
                hg[j].wait()
                hsc[j] = pltpu.async_copy(
                    rows.at[j % 2], acc.at[didx.at[j]], ss, add=True)
            hsc[GB - 1].wait()
            return carry

        lax.fori_loop(0, JSTEPS // GB, body, 0)
        plsc.subcore_barrier()
        pltpu.sync_copy(acc.at[pl.ds(s * ZROWS, ZROWS)],
                        out_hbm.at[pl.ds(s * ZROWS, ZROWS)])

    return k(ep, dst2d, z128)


def _tc_combine_body(nodes_b, q_b, p_b, s32_b, r32_b,
                     m1, m2, m3, m4, m5, m6, m7, cst, out_b):
    nb = nodes_b[...]
    od = s32_b[:, 16:17]
    ind = r32_b[:, 16:17]
    f32 = jnp.float32
    y = jnp.dot(nb, m1[...], preferred_element_type=f32,
                 precision=lax.Precision.HIGHEST)
    y += jnp.dot(q_b[...], m2[...], preferred_element_type=f32,
                 precision=lax.Precision.HIGHEST)
    y += jnp.dot(p_b[...], m3[...], preferred_element_type=f32,
                 precision=lax.Precision.HIGHEST)
    y += jnp.dot(s32_b[...], m4[...], preferred_element_type=f32,
                 precision=lax.Precision.HIGHEST)
    y += jnp.dot(r32_b[...], m5[...], preferred_element_type=f32,
                 precision=lax.Precision.HIGHEST)
    y += od * jnp.dot(nb, m6[...], preferred_element_type=f32,
                 precision=lax.Precision.HIGHEST)
    y += ind * jnp.dot(nb, m7[...], preferred_element_type=f32,
                 precision=lax.Precision.HIGHEST)
    out_b[...] = y + cst[...]


def kernel(nodes, edges, senders, receivers, globals_, W_ne, b_ne, W_ee, b_ee,
           W_eu, b_eu, W_nu, b_nu, W_gu, b_gu, W_out, b_out):
    g0 = globals_[0]
    We, Ws, Wr, Wg = W_eu[:128], W_eu[128:256], W_eu[256:384], W_eu[384:]
    Wnu_n, Wnu_s, Wnu_r, Wnu_g = (W_nu[:128], W_nu[128:256],
                                  W_nu[256:384], W_nu[384:])

    mm = lambda a, b: jnp.matmul(a, b, precision=lax.Precision.HIGHEST)
    Kas = mm(W_ne, Ws)
    Kbs = mm(W_ne, Wr)
    Ke = mm(W_ee, We)
    c0 = mm(b_ee, We) + mm(g0, Wg) + b_eu
    cS = c0 + mm(b_ne, Ws) + mm(b_ne, Wr)
    P_n = mm(Wnu_n, W_out)
    P_s = mm(Wnu_s, W_out)
    P_r = mm(Wnu_r, W_out)

    def pad8(m):
        return jnp.pad(m, ((0, 0), (0, 8 - m.shape[1])))

    M1 = pad8(mm(W_ne, P_n))
    M2 = pad8(mm(Kbs, P_s))
    M3 = pad8(mm(Kas, P_r))
    M4 = pad8(jnp.zeros((32, 5), jnp.float32).at[:16].set(mm(Ke, P_s))
              .at[16].set(mm(cS, P_s)))
    M5 = pad8(jnp.zeros((32, 5), jnp.float32).at[:16].set(mm(Ke, P_r))
              .at[16].set(mm(cS, P_r)))
    M6 = pad8(mm(Kas, P_s))
    M7 = pad8(mm(Kbs, P_r))
    const = jnp.pad(mm(b_ne, P_n) + mm(mm(g0, Wnu_g) + b_nu, W_out) + b_out,
                    (0, 3))[None, :]

    nodes_pad = jnp.pad(nodes, ((0, NPAD - N), (0, 0)))
    edges_plus = jnp.concatenate(
        [edges, jnp.ones((E, 1), jnp.float32),
         jnp.zeros((E, 111), jnp.float32)], axis=1)
    edges_plus = jnp.pad(edges_plus, ((0, EPAD - E), (0, 0)))
    pad_idx = jnp.full((EPAD - E,), N, jnp.int32)
    sidx2d = jnp.concatenate([senders, pad_idx]).reshape(JSTEPS * TILES, TRANS)
    ridx2d = jnp.concatenate([receivers, pad_idx]).reshape(JSTEPS * TILES,
                                                           TRANS)
    z128 = jnp.zeros((ZROWS, 128), jnp.float32)

    qraw = _sc_gather_scatter(nodes_pad, ridx2d, sidx2d, z128)
    praw = _sc_gather_scatter(nodes_pad, sidx2d, ridx2d, z128)
    s32 = _sc_linear_scatter(edges_plus, sidx2d, z128)
    r32 = _sc_linear_scatter(edges_plus, ridx2d, z128)
    qraw, praw = qraw[:N], praw[:N]
    s32, r32 = s32[:N, :32], r32[:N, :32]

    blk = 2000
    grid = N // blk
    row_spec = lambda w: pl.BlockSpec((blk, w), lambda i: (i, 0))
    full_spec = lambda shape: pl.BlockSpec(shape, lambda i: (0, 0))
    out8 = pl.pallas_call(
        _tc_combine_body,
        grid=(grid,),
        in_specs=[row_spec(128), row_spec(128), row_spec(128),
                  row_spec(32), row_spec(32),
                  full_spec((128, 8)), full_spec((128, 8)), full_spec((128, 8)),
                  full_spec((32, 8)), full_spec((32, 8)),
                  full_spec((128, 8)), full_spec((128, 8)), full_spec((1, 8))],
        out_specs=row_spec(8),
        out_shape=jax.ShapeDtypeStruct((N, 8), jnp.float32),
    )(nodes, qraw, praw, s32, r32, M1, M2, M3, M4, M5, M6, M7, const)
    return out8[:, :5]


# speedup vs baseline: 1.1214x; 1.1214x over previous
import functools

import jax
import jax.numpy as jnp
from jax import lax
from jax.experimental import pallas as pl
from jax.experimental.pallas import tpu as pltpu
from jax.experimental.pallas import tpu_sc as plsc

N = 10000
E = 320000
NPAD = 10112
TILES = 16
TRANS = 128
JSTEPS = 160
EPT = TRANS * JSTEPS
EPAD = EPT * TILES
ZROWS = NPAD // TILES
GB = 16


def _sc_node_passes(nodes_pad, sidx2d, ridx2d, z128):
    mesh = plsc.VectorSubcoreMesh(core_axis_name="c", subcore_axis_name="s",
                                  num_cores=1)

    @functools.partial(
        pl.kernel,
        mesh=mesh,
        out_type=(
            jax.ShapeDtypeStruct((NPAD, 128), jnp.float32),
            jax.ShapeDtypeStruct((NPAD, 128), jnp.float32),
        ),
        scratch_types=[
            pltpu.VMEM((GB, TRANS), jnp.int32),
            pltpu.VMEM((GB, TRANS), jnp.int32),
            pltpu.VMEM((2, TRANS, 128), jnp.float32),
            pltpu.VMEM_SHARED((NPAD, 128), jnp.float32),
            pltpu.SemaphoreType.DMA,
            pltpu.SemaphoreType.DMA,
        ],
    )
    def k(tab_hbm, s_hbm, r_hbm, z128_hbm, qraw_hbm, praw_hbm,
          gidx, didx, rows, acc, sg, ss):
        s = lax.axis_index("s")

        def run_pass(gat_hbm, dst_hbm, out_hbm):
            pltpu.sync_copy(z128_hbm, acc.at[pl.ds(s * ZROWS, ZROWS)])
            plsc.subcore_barrier()

            def body(g, carry):
                grow = s * JSTEPS + g * GB
                pltpu.sync_copy(gat_hbm.at[pl.ds(grow, GB)], gidx)
                pltpu.sync_copy(dst_hbm.at[pl.ds(grow, GB)], didx)
                hg = {}
                hsc = {}
                hg[0] = pltpu.async_copy(tab_hbm.at[gidx.at[0]],
                                         rows.at[0], sg)
                hg[1] = pltpu.async_copy(tab_hbm.at[gidx.at[1]],
                                         rows.at[1], sg)
                for j in range(GB):
                    if j >= 1:
                        hsc[j - 1].wait()
                    if 1 <= j and j + 1 < GB:
                        hg[j + 1] = pltpu.async_copy(
                            tab_hbm.at[gidx.at[j + 1]],
                            rows.at[(j + 1) % 2], sg)
                    hg[j].wait()
                    hsc[j] = pltpu.async_copy(
                        rows.at[j % 2], acc.at[didx.at[j]], ss, add=True)
                hsc[GB - 1].wait()
                return carry

            lax.fori_loop(0, JSTEPS // GB, body, 0)
            plsc.subcore_barrier()
            pltpu.sync_copy(acc.at[pl.ds(s * ZROWS, ZROWS)],
                            out_hbm.at[pl.ds(s * ZROWS, ZROWS)])

        run_pass(r_hbm, s_hbm, qraw_hbm)
        run_pass(s_hbm, r_hbm, praw_hbm)

    return k(nodes_pad, sidx2d, ridx2d, z128)


def _sc_edge_pass(ep128, sidx2d, ridx2d, z128):
    mesh = plsc.VectorSubcoreMesh(core_axis_name="c", subcore_axis_name="s",
                                  num_cores=1)

    @functools.partial(
        pl.kernel,
        mesh=mesh,
        out_type=(
            jax.ShapeDtypeStruct((NPAD, 128), jnp.float32),
            jax.ShapeDtypeStruct((NPAD, 128), jnp.float32),
        ),
        scratch_types=[
            pltpu.VMEM((GB, TRANS), jnp.int32),
            pltpu.VMEM((2, TRANS, 128), jnp.float32),
            pltpu.VMEM_SHARED((NPAD, 128), jnp.float32),
            pltpu.SemaphoreType.DMA,
            pltpu.SemaphoreType.DMA,
        ],
    )
    def k(ep_hbm, s_hbm, r_hbm, z128_hbm, s32_hbm, r32_hbm,
          didx, rows, acc, sg, ss):
        s = lax.axis_index("s")

        def run_pass(dst_hbm, out_hbm):
            pltpu.sync_copy(z128_hbm, acc.at[pl.ds(s * ZROWS, ZROWS)])
            plsc.subcore_barrier()

            def body(g, carry):
                grow = s * JSTEPS + g * GB
                pltpu.sync_copy(dst_hbm.at[pl.ds(grow, GB)], didx)
                base = s * EPT + g * GB * TRANS
                hg = {}
                hsc = {}
                hg[0] = pltpu.async_copy(
                    ep_hbm.at[pl.ds(base, TRANS)], rows.at[0], sg)
                hg[1] = pltpu.async_copy(
                    ep_hbm.at[pl.ds(base + TRANS, TRANS)], rows.at[1], sg)
                for j in range(GB):
                    if j >= 1:
                        hsc[j - 1].wait()
                    if 1 <= j and j + 1 < GB:
                        hg[j + 1] = pltpu.async_copy(
                            ep_hbm.at[pl.ds(base + (j + 1) * TRANS, TRANS)],
                            rows.at[(j + 1) % 2], sg)
                    hg[j].wait()
                    hsc[j] = pltpu.async_copy(
                        rows.at[j % 2], acc.at[didx.at[j]], ss, add=True)
                hsc[GB - 1].wait()
                return carry

            lax.fori_loop(0, JSTEPS // GB, body, 0)
            plsc.subcore_barrier()
            pltpu.sync_copy(acc.at[pl.ds(s * ZROWS, ZROWS)],
                            out_hbm.at[pl.ds(s * ZROWS, ZROWS)])

        run_pass(s_hbm, s32_hbm)
        run_pass(r_hbm, r32_hbm)

    return k(ep128, sidx2d, ridx2d, z128)


def _tc_combine_body(nodes_b, q_b, p_b, s32_b, r32_b,
                     m1, m2, m3, m4, m5, m6, m7, cst, out_b):
    nb = nodes_b[...]
    od = s32_b[:, 16:17]
    ind = r32_b[:, 16:17]
    f32 = jnp.float32
    y = jnp.dot(nb, m1[...], preferred_element_type=f32,
                 precision=lax.Precision.HIGHEST)
    y += jnp.dot(q_b[...], m2[...], preferred_element_type=f32,
                 precision=lax.Precision.HIGHEST)
    y += jnp.dot(p_b[...], m3[...], preferred_element_type=f32,
                 precision=lax.Precision.HIGHEST)
    y += jnp.dot(s32_b[...], m4[...], preferred_element_type=f32,
                 precision=lax.Precision.HIGHEST)
    y += jnp.dot(r32_b[...], m5[...], preferred_element_type=f32,
                 precision=lax.Precision.HIGHEST)
    y += od * jnp.dot(nb, m6[...], preferred_element_type=f32,
                 precision=lax.Precision.HIGHEST)
    y += ind * jnp.dot(nb, m7[...], preferred_element_type=f32,
                 precision=lax.Precision.HIGHEST)
    out_b[...] = y + cst[...]


def kernel(nodes, edges, senders, receivers, globals_, W_ne, b_ne, W_ee, b_ee,
           W_eu, b_eu, W_nu, b_nu, W_gu, b_gu, W_out, b_out):
    g0 = globals_[0]
    We, Ws, Wr, Wg = W_eu[:128], W_eu[128:256], W_eu[256:384], W_eu[384:]
    Wnu_n, Wnu_s, Wnu_r, Wnu_g = (W_nu[:128], W_nu[128:256],
                                  W_nu[256:384], W_nu[384:])

    mm = lambda a, b: jnp.matmul(a, b, precision=lax.Precision.HIGHEST)
    Kas = mm(W_ne, Ws)
    Kbs = mm(W_ne, Wr)
    Ke = mm(W_ee, We)
    c0 = mm(b_ee, We) + mm(g0, Wg) + b_eu
    cS = c0 + mm(b_ne, Ws) + mm(b_ne, Wr)
    P_n = mm(Wnu_n, W_out)
    P_s = mm(Wnu_s, W_out)
    P_r = mm(Wnu_r, W_out)

    def pad8(m):
        return jnp.pad(m, ((0, 0), (0, 8 - m.shape[1])))

    M1 = pad8(mm(W_ne, P_n))
    M2 = pad8(mm(Kbs, P_s))
    M3 = pad8(mm(Kas, P_r))
    M4 = pad8(jnp.zeros((32, 5), jnp.float32).at[:16].set(mm(Ke, P_s))
              .at[16].set(mm(cS, P_s)))
    M5 = pad8(jnp.zeros((32, 5), jnp.float32).at[:16].set(mm(Ke, P_r))
              .at[16].set(mm(cS, P_r)))
    M6 = pad8(mm(Kas, P_s))
    M7 = pad8(mm(Kbs, P_r))
    const = jnp.pad(mm(b_ne, P_n) + mm(mm(g0, Wnu_g) + b_nu, W_out) + b_out,
                    (0, 3))[None, :]

    nodes_pad = jnp.pad(nodes, ((0, NPAD - N), (0, 0)))
    edges_plus = jnp.concatenate(
        [edges, jnp.ones((E, 1), jnp.float32),
         jnp.zeros((E, 111), jnp.float32)], axis=1)
    edges_plus = jnp.pad(edges_plus, ((0, EPAD - E), (0, 0)))
    pad_idx = jnp.full((EPAD - E,), N, jnp.int32)
    sidx2d = jnp.concatenate([senders, pad_idx]).reshape(JSTEPS * TILES, TRANS)
    ridx2d = jnp.concatenate([receivers, pad_idx]).reshape(JSTEPS * TILES,
                                                           TRANS)
    z128 = jnp.zeros((ZROWS, 128), jnp.float32)

    qraw, praw = _sc_node_passes(nodes_pad, sidx2d, ridx2d, z128)
    s32, r32 = _sc_edge_pass(edges_plus, sidx2d, ridx2d, z128)
    qraw, praw = qraw[:N], praw[:N]
    s32, r32 = s32[:N, :32], r32[:N, :32]

    blk = 2000
    grid = N // blk
    row_spec = lambda w: pl.BlockSpec((blk, w), lambda i: (i, 0))
    full_spec = lambda shape: pl.BlockSpec(shape, lambda i: (0, 0))
    out8 = pl.pallas_call(
        _tc_combine_body,
        grid=(grid,),
        in_specs=[row_spec(128), row_spec(128), row_spec(128),
                  row_spec(32), row_spec(32),
                  full_spec((128, 8)), full_spec((128, 8)), full_spec((128, 8)),
                  full_spec((32, 8)), full_spec((32, 8)),
                  full_spec((128, 8)), full_spec((128, 8)), full_spec((1, 8))],
        out_specs=row_spec(8),
        out_shape=jax.ShapeDtypeStruct((N, 8), jnp.float32),
    )(nodes, qraw, praw, s32, r32, M1, M2, M3, M4, M5, M6, M7, const)
    return out8[:, :5]
